# baseline (speedup 1.0000x reference)
import functools

import jax
import jax.numpy as jnp
from jax import lax
from jax.experimental import pallas as pl
from jax.experimental.pallas import tpu as pltpu
from jax.experimental.pallas import tpu_sc as plsc

NC = 2
NS = 16
NW = NC * NS
L = 16
G = 128


def _lane_perm(v, idx):
    return lax.gather(
        v, idx[:, None],
        dimension_numbers=lax.GatherDimensionNumbers(
            offset_dims=(), collapsed_slice_dims=(0,), start_index_map=(0,)),
        slice_sizes=(1,),
        mode=lax.GatherScatterMode.PROMISE_IN_BOUNDS)


def _lane_allsum(v):
    for sh in (8, 4, 2, 1):
        idx = lax.iota(jnp.int32, L) ^ sh
        v = v + _lane_perm(v, idx)
    return v


def _rsqrt(v):
    vi = lax.bitcast_convert_type(v, jnp.int32)
    yi = jnp.int32(0x5F3759DF) - lax.shift_right_logical(vi, 1)
    y = lax.bitcast_convert_type(yi, jnp.float32)
    half_v = v * 0.5
    for _ in range(3):
        y = y * (1.5 - half_v * y * y)
    return y


def kernel(x, word_table, pos_table, chan_table, gamma, beta):
    C, B, S = x.shape
    V, H = word_table.shape
    N = C * B * S
    nvr = H // L
    rows_per_w = N // NW
    n_chunks = rows_per_w // G
    assert rows_per_w % G == 0 and H % L == 0

    pc = (chan_table[:, None, :] + pos_table[None, :, :]).reshape(C * S * H)
    xf = x.reshape(N)

    mesh = plsc.VectorSubcoreMesh(
        core_axis_name="c", subcore_axis_name="s", num_cores=NC, num_subcores=NS
    )

    @functools.partial(
        pl.kernel,
        out_type=jax.ShapeDtypeStruct((N, H), jnp.float32),
        mesh=mesh,
        scratch_types=[
            pltpu.VMEM((C * S * H,), jnp.float32),
            pltpu.VMEM((G,), jnp.int32),
            pltpu.VMEM((G, H), jnp.float32),
            pltpu.VMEM((H,), jnp.float32),
            pltpu.VMEM((H,), jnp.float32),
            pltpu.SemaphoreType.DMA,
        ],
    )
    def sc_kernel(xf_hbm, wt_hbm, pc_hbm, g_hbm, b_hbm, out_hbm,
                  pc_v, idx_v, rows_v, g_v, b_v, sem):
        wid = lax.axis_index("s") * NC + lax.axis_index("c")
        base0 = wid * rows_per_w
        pltpu.sync_copy(pc_hbm, pc_v)
        pltpu.sync_copy(g_hbm, g_v)
        pltpu.sync_copy(b_hbm, b_v)
        gs = [g_v[pl.ds(j * L, L)] for j in range(nvr)]
        bs = [b_v[pl.ds(j * L, L)] for j in range(nvr)]

        @pl.loop(0, n_chunks)
        def _chunk(g):
            base = base0 + g * G
            pltpu.sync_copy(xf_hbm.at[pl.ds(base, G)], idx_v)
            pltpu.async_copy(wt_hbm.at[idx_v], rows_v, sem).wait()

            @pl.loop(0, G)
            def _row(i):
                r = base + i
                s_ = lax.rem(r, S)
                c_ = lax.div(r, B * S)
                off = (c_ * S + s_) * H
                xs = []
                for j in range(nvr):
                    w = rows_v[i, pl.ds(j * L, L)]
                    p = pc_v[pl.ds(off + j * L, L)]
                    xs.append(w + p)
                ssum = xs[0]
                for j in range(1, nvr):
                    ssum = ssum + xs[j]
                qsum = xs[0] * xs[0]
                for j in range(1, nvr):
                    qsum = qsum + xs[j] * xs[j]
                mean = _lane_allsum(ssum) * (1.0 / H)
                ex2 = _lane_allsum(qsum) * (1.0 / H)
                var = ex2 - mean * mean
                a = _rsqrt(var + 1e-5)
                for j in range(nvr):
                    rows_v[i, pl.ds(j * L, L)] = (xs[j] - mean) * a * gs[j] + bs[j]

            pltpu.sync_copy(rows_v, out_hbm.at[pl.ds(base, G)])

    out = sc_kernel(xf, word_table, pc, gamma, beta)
    return out.reshape(C, B, S, H)


# speedup vs baseline: 1.3572x; 1.3572x over previous
import functools

import jax
import jax.numpy as jnp
from jax import lax
from jax.experimental import pallas as pl
from jax.experimental.pallas import tpu as pltpu
from jax.experimental.pallas import tpu_sc as plsc

NC = 2
NS = 16
NW = NC * NS
L = 16
G = 128
NBUF = 3


def _lane_perm(v, idx):
    return lax.gather(
        v, idx[:, None],
        dimension_numbers=lax.GatherDimensionNumbers(
            offset_dims=(), collapsed_slice_dims=(0,), start_index_map=(0,)),
        slice_sizes=(1,),
        mode=lax.GatherScatterMode.PROMISE_IN_BOUNDS)


def _lane_allsum(v):
    for sh in (8, 4, 2, 1):
        idx = lax.iota(jnp.int32, L) ^ sh
        v = v + _lane_perm(v, idx)
    return v


def _rsqrt(v):
    vi = lax.bitcast_convert_type(v, jnp.int32)
    yi = jnp.int32(0x5F3759DF) - lax.shift_right_logical(vi, 1)
    y = lax.bitcast_convert_type(yi, jnp.float32)
    half_v = v * 0.5
    for _ in range(3):
        y = y * (1.5 - half_v * y * y)
    return y


def kernel(x, word_table, pos_table, chan_table, gamma, beta):
    C, B, S = x.shape
    V, H = word_table.shape
    N = C * B * S
    nvr = H // L
    rows_per_w = N // NW
    n_chunks = rows_per_w // G
    assert rows_per_w % G == 0 and H % L == 0

    pc = (chan_table[:, None, :] + pos_table[None, :, :]).reshape(C * S * H)
    xf = x.reshape(N)

    mesh = plsc.VectorSubcoreMesh(
        core_axis_name="c", subcore_axis_name="s", num_cores=NC, num_subcores=NS
    )

    @functools.partial(
        pl.kernel,
        out_type=jax.ShapeDtypeStruct((N, H), jnp.float32),
        mesh=mesh,
        scratch_types=[
            pltpu.VMEM((C * S * H,), jnp.float32),
            pltpu.VMEM((NBUF, G), jnp.int32),
            pltpu.VMEM((NBUF, G, H), jnp.float32),
            pltpu.VMEM((H,), jnp.float32),
            pltpu.VMEM((H,), jnp.float32),
        ] + [pltpu.SemaphoreType.DMA] * (3 * NBUF),
    )
    def sc_kernel(xf_hbm, wt_hbm, pc_hbm, g_hbm, b_hbm, out_hbm,
                  pc_v, idx_v, rows_v, g_v, b_v, *sems):
        gsem = sems[0:NBUF]
        osem = sems[NBUF:2 * NBUF]
        isem = sems[2 * NBUF:3 * NBUF]
        wid = lax.axis_index("s") * NC + lax.axis_index("c")
        base0 = wid * rows_per_w
        pltpu.sync_copy(pc_hbm, pc_v)
        pltpu.sync_copy(g_hbm, g_v)
        pltpu.sync_copy(b_hbm, b_v)
        gs = [g_v[pl.ds(j * L, L)] for j in range(nvr)]
        bs = [b_v[pl.ds(j * L, L)] for j in range(nvr)]

        def compute(g, b):
            base = base0 + g * G

            @pl.loop(0, G, unroll=2)
            def _row(i):
                r = base + i
                s_ = lax.rem(r, S)
                c_ = lax.div(r, B * S)
                off = (c_ * S + s_) * H
                xs = []
                for j in range(nvr):
                    w = rows_v[b, i, pl.ds(j * L, L)]
                    p = pc_v[pl.ds(off + j * L, L)]
                    xs.append(w + p)
                ssum = xs[0]
                for j in range(1, nvr):
                    ssum = ssum + xs[j]
                qsum = xs[0] * xs[0]
                for j in range(1, nvr):
                    qsum = qsum + xs[j] * xs[j]
                mean = _lane_allsum(ssum) * (1.0 / H)
                ex2 = _lane_allsum(qsum) * (1.0 / H)
                var = ex2 - mean * mean
                a = _rsqrt(var + 1e-5)
                for j in range(nvr):
                    rows_v[b, i, pl.ds(j * L, L)] = \
                        (xs[j] - mean) * a * gs[j] + bs[j]

        def idx_copy(g, b):
            pltpu.async_copy(
                xf_hbm.at[pl.ds(base0 + g * G, G)], idx_v.at[b], isem[b])

        def gather(b):
            pltpu.async_copy(wt_hbm.at[idx_v.at[b]], rows_v.at[b], gsem[b])

        for b in range(2):
            pltpu.sync_copy(xf_hbm.at[pl.ds(base0 + b * G, G)], idx_v.at[b])
            gather(b)
        idx_copy(2, 2 % NBUF)

        @pl.loop(0, n_chunks, step=NBUF)
        def _outer(gbase):
            for b in range(NBUF):
                g = gbase + b
                b2 = (b + 2) % NBUF
                pltpu.make_async_copy(
                    wt_hbm.at[idx_v.at[b]], rows_v.at[b], gsem[b]).wait()
                compute(g, b)
                pltpu.async_copy(
                    rows_v.at[b], out_hbm.at[pl.ds(base0 + g * G, G)], osem[b])

                @pl.when(g + NBUF < n_chunks)
                def _():
                    idx_copy(g + NBUF, b)

                @pl.when(g >= 1)
                def _():
                    pltpu.make_async_copy(
                        rows_v.at[b2],
                        out_hbm.at[pl.ds(base0, G)], osem[b2]).wait()

                @pl.when(g + 2 < n_chunks)
                def _():
                    pltpu.make_async_copy(
                        xf_hbm.at[pl.ds(base0, G)], idx_v.at[b2],
                        isem[b2]).wait()
                    gather(b2)

        bl = (n_chunks - 1) % NBUF
        pltpu.make_async_copy(
            rows_v.at[bl], out_hbm.at[pl.ds(base0, G)], osem[bl]).wait()

    out = sc_kernel(xf, word_table, pc, gamma, beta)
    return out.reshape(C, B, S, H)


# speedup vs baseline: 1.5213x; 1.1209x over previous
import functools

import jax
import jax.numpy as jnp
from jax import lax
from jax.experimental import pallas as pl
from jax.experimental.pallas import tpu as pltpu
from jax.experimental.pallas import tpu_sc as plsc

NC = 2
NS = 16
NW = NC * NS
L = 16
G = 128
NBUF = 3


def _lane_perm(v, idx):
    return lax.gather(
        v, idx[:, None],
        dimension_numbers=lax.GatherDimensionNumbers(
            offset_dims=(), collapsed_slice_dims=(0,), start_index_map=(0,)),
        slice_sizes=(1,),
        mode=lax.GatherScatterMode.PROMISE_IN_BOUNDS)


def _lane_allsum(v):
    for sh in (8, 4, 2, 1):
        idx = lax.iota(jnp.int32, L) ^ sh
        v = v + _lane_perm(v, idx)
    return v


def _rsqrt(v):
    vi = lax.bitcast_convert_type(v, jnp.int32)
    yi = jnp.int32(0x5F3759DF) - lax.shift_right_logical(vi, 1)
    y = lax.bitcast_convert_type(yi, jnp.float32)
    half_v = v * 0.5
    for _ in range(2):
        y = y * (1.5 - half_v * y * y)
    return y


def kernel(x, word_table, pos_table, chan_table, gamma, beta):
    C, B, S = x.shape
    V, H = word_table.shape
    N = C * B * S
    nvr = H // L
    rows_per_w = N // NW
    n_chunks = rows_per_w // G
    assert rows_per_w % G == 0 and H % L == 0

    pc = (chan_table[:, None, :] + pos_table[None, :, :]).reshape(C * S * H)
    xf = x.reshape(N)

    mesh = plsc.VectorSubcoreMesh(
        core_axis_name="c", subcore_axis_name="s", num_cores=NC, num_subcores=NS
    )

    @functools.partial(
        pl.kernel,
        out_type=jax.ShapeDtypeStruct((N, H), jnp.float32),
        mesh=mesh,
        scratch_types=[
            pltpu.VMEM((C * S * H,), jnp.float32),
            pltpu.VMEM((NBUF, G), jnp.int32),
            pltpu.VMEM((NBUF, G, H), jnp.float32),
            pltpu.VMEM((H,), jnp.float32),
            pltpu.VMEM((H,), jnp.float32),
        ] + [pltpu.SemaphoreType.DMA] * (3 * NBUF),
    )
    def sc_kernel(xf_hbm, wt_hbm, pc_hbm, g_hbm, b_hbm, out_hbm,
                  pc_v, idx_v, rows_v, g_v, b_v, *sems):
        gsem = sems[0:NBUF]
        osem = sems[NBUF:2 * NBUF]
        isem = sems[2 * NBUF:3 * NBUF]
        wid = lax.axis_index("s") * NC + lax.axis_index("c")
        base0 = wid * rows_per_w
        pltpu.sync_copy(pc_hbm, pc_v)
        pltpu.sync_copy(g_hbm, g_v)
        pltpu.sync_copy(b_hbm, b_v)
        gs = [g_v[pl.ds(j * L, L)] for j in range(nvr)]
        bs = [b_v[pl.ds(j * L, L)] for j in range(nvr)]

        def compute(g, b):
            base = base0 + g * G

            @pl.loop(0, G, unroll=4)
            def _row(i):
                r = base + i
                s_ = lax.rem(r, S)
                c_ = lax.div(r, B * S)
                off = (c_ * S + s_) * H
                xs = []
                for j in range(nvr):
                    w = rows_v[b, i, pl.ds(j * L, L)]
                    p = pc_v[pl.ds(off + j * L, L)]
                    xs.append(w + p)
                ssum = xs[0]
                for j in range(1, nvr):
                    ssum = ssum + xs[j]
                qsum = xs[0] * xs[0]
                for j in range(1, nvr):
                    qsum = qsum + xs[j] * xs[j]
                mean = _lane_allsum(ssum) * (1.0 / H)
                ex2 = _lane_allsum(qsum) * (1.0 / H)
                var = ex2 - mean * mean
                a = _rsqrt(var + 1e-5)
                for j in range(nvr):
                    rows_v[b, i, pl.ds(j * L, L)] = \
                        (xs[j] - mean) * a * gs[j] + bs[j]

        def idx_copy(g, b):
            pltpu.async_copy(
                xf_hbm.at[pl.ds(base0 + g * G, G)], idx_v.at[b], isem[b])

        def gather(b):
            pltpu.async_copy(wt_hbm.at[idx_v.at[b]], rows_v.at[b], gsem[b])

        for b in range(2):
            pltpu.sync_copy(xf_hbm.at[pl.ds(base0 + b * G, G)], idx_v.at[b])
            gather(b)
        idx_copy(2, 2 % NBUF)

        @pl.loop(0, n_chunks, step=NBUF)
        def _outer(gbase):
            for b in range(NBUF):
                g = gbase + b
                b2 = (b + 2) % NBUF
                pltpu.make_async_copy(
                    wt_hbm.at[idx_v.at[b]], rows_v.at[b], gsem[b]).wait()
                compute(g, b)
                pltpu.async_copy(
                    rows_v.at[b], out_hbm.at[pl.ds(base0 + g * G, G)], osem[b])

                @pl.when(g + NBUF < n_chunks)
                def _():
                    idx_copy(g + NBUF, b)

                @pl.when(g >= 1)
                def _():
                    pltpu.make_async_copy(
                        rows_v.at[b2],
                        out_hbm.at[pl.ds(base0, G)], osem[b2]).wait()

                @pl.when(g + 2 < n_chunks)
                def _():
                    pltpu.make_async_copy(
                        xf_hbm.at[pl.ds(base0, G)], idx_v.at[b2],
                        isem[b2]).wait()
                    gather(b2)

        bl = (n_chunks - 1) % NBUF
        pltpu.make_async_copy(
            rows_v.at[bl], out_hbm.at[pl.ds(base0, G)], osem[bl]).wait()

    out = sc_kernel(xf, word_table, pc, gamma, beta)
    return out.reshape(C, B, S, H)


# speedup vs baseline: 2.8853x; 1.8967x over previous
import functools

import jax
import jax.numpy as jnp
from jax import lax
from jax.experimental import pallas as pl
from jax.experimental.pallas import tpu as pltpu
from jax.experimental.pallas import tpu_sc as plsc

NC = 2
NS = 16
NW = NC * NS
L = 16
G = 128
NBUF = 3


def _lane_perm(v, idx):
    return lax.gather(
        v, idx[:, None],
        dimension_numbers=lax.GatherDimensionNumbers(
            offset_dims=(), collapsed_slice_dims=(0,), start_index_map=(0,)),
        slice_sizes=(1,),
        mode=lax.GatherScatterMode.PROMISE_IN_BOUNDS)


def _lane_allsum(v):
    for sh in (8, 4, 2, 1):
        idx = lax.iota(jnp.int32, L) ^ sh
        v = v + _lane_perm(v, idx)
    return v


def _rsqrt(v):
    vi = lax.bitcast_convert_type(v, jnp.int32)
    yi = jnp.int32(0x5F3759DF) - lax.shift_right_logical(vi, 1)
    y = lax.bitcast_convert_type(yi, jnp.float32)
    half_v = v * 0.5
    for _ in range(2):
        y = y * (1.5 - half_v * y * y)
    return y


def kernel(x, word_table, pos_table, chan_table, gamma, beta):
    C, B, S = x.shape
    V, H = word_table.shape
    N = C * B * S
    nvr = H // L
    rows_per_w = N // NW
    n_chunks = rows_per_w // G
    assert rows_per_w % G == 0 and H % L == 0

    pc = (chan_table[:, None, :] + pos_table[None, :, :]).reshape(C * S * H)
    xf = x.reshape(N)

    mesh = plsc.VectorSubcoreMesh(
        core_axis_name="c", subcore_axis_name="s", num_cores=NC, num_subcores=NS
    )

    @functools.partial(
        pl.kernel,
        out_type=jax.ShapeDtypeStruct((N, H), jnp.float32),
        mesh=mesh,
        scratch_types=[
            pltpu.VMEM((C * S * H,), jnp.float32),
            pltpu.VMEM((NBUF, G), jnp.int32),
            pltpu.VMEM((NBUF, G, H), jnp.float32),
            pltpu.VMEM((H,), jnp.float32),
            pltpu.VMEM((H,), jnp.float32),
        ] + [pltpu.SemaphoreType.DMA] * (3 * NBUF),
    )
    def sc_kernel(xf_hbm, wt_hbm, pc_hbm, g_hbm, b_hbm, out_hbm,
                  pc_v, idx_v, rows_v, g_v, b_v, *sems):
        gsem = sems[0:NBUF]
        osem = sems[NBUF:2 * NBUF]
        isem = sems[2 * NBUF:3 * NBUF]
        wid = lax.axis_index("s") * NC + lax.axis_index("c")
        base0 = wid * rows_per_w
        pltpu.sync_copy(pc_hbm, pc_v)
        pltpu.sync_copy(g_hbm, g_v)
        pltpu.sync_copy(b_hbm, b_v)
        gs = [g_v[pl.ds(j * L, L)] for j in range(nvr)]
        bs = [b_v[pl.ds(j * L, L)] for j in range(nvr)]

        def compute(g, b):
            base = base0 + g * G

            @plsc.parallel_loop(0, G, unroll=4)
            def _row(i):
                r = base + i
                s_ = lax.rem(r, S)
                c_ = lax.div(r, B * S)
                off = (c_ * S + s_) * H
                xs = []
                for j in range(nvr):
                    w = rows_v[b, i, pl.ds(j * L, L)]
                    p = pc_v[pl.ds(off + j * L, L)]
                    xs.append(w + p)
                ssum = xs[0]
                for j in range(1, nvr):
                    ssum = ssum + xs[j]
                qsum = xs[0] * xs[0]
                for j in range(1, nvr):
                    qsum = qsum + xs[j] * xs[j]
                mean = _lane_allsum(ssum) * (1.0 / H)
                ex2 = _lane_allsum(qsum) * (1.0 / H)
                var = ex2 - mean * mean
                a = _rsqrt(var + 1e-5)
                for j in range(nvr):
                    rows_v[b, i, pl.ds(j * L, L)] = \
                        (xs[j] - mean) * a * gs[j] + bs[j]

        def idx_copy(g, b):
            pltpu.async_copy(
                xf_hbm.at[pl.ds(base0 + g * G, G)], idx_v.at[b], isem[b])

        def gather(b):
            pltpu.async_copy(wt_hbm.at[idx_v.at[b]], rows_v.at[b], gsem[b])

        for b in range(2):
            pltpu.sync_copy(xf_hbm.at[pl.ds(base0 + b * G, G)], idx_v.at[b])
            gather(b)
        idx_copy(2, 2 % NBUF)

        @pl.loop(0, n_chunks, step=NBUF)
        def _outer(gbase):
            for b in range(NBUF):
                g = gbase + b
                b2 = (b + 2) % NBUF
                pltpu.make_async_copy(
                    wt_hbm.at[idx_v.at[b]], rows_v.at[b], gsem[b]).wait()
                compute(g, b)
                pltpu.async_copy(
                    rows_v.at[b], out_hbm.at[pl.ds(base0 + g * G, G)], osem[b])

                @pl.when(g + NBUF < n_chunks)
                def _():
                    idx_copy(g + NBUF, b)

                @pl.when(g >= 1)
                def _():
                    pltpu.make_async_copy(
                        rows_v.at[b2],
                        out_hbm.at[pl.ds(base0, G)], osem[b2]).wait()

                @pl.when(g + 2 < n_chunks)
                def _():
                    pltpu.make_async_copy(
                        xf_hbm.at[pl.ds(base0, G)], idx_v.at[b2],
                        isem[b2]).wait()
                    gather(b2)

        bl = (n_chunks - 1) % NBUF
        pltpu.make_async_copy(
            rows_v.at[bl], out_hbm.at[pl.ds(base0, G)], osem[bl]).wait()

    out = sc_kernel(xf, word_table, pc, gamma, beta)
    return out.reshape(C, B, S, H)
